# baseline (speedup 1.0000x reference)
import jax
import jax.numpy as jnp
from jax import lax
from jax.experimental import pallas as pl
from jax.experimental.pallas import tpu as pltpu
from jax.experimental.pallas import tpu_sc as plsc

_N = 10000
_E = 320000
_D = 128
_NC = 2
_NS = 16
_C = 128
_K = 80
_EPAD = _NC * _NS * _K * _C
_NPAD = 10240
_RPT = _NPAD // _NS
_PAD_SRC = _N
_PAD_DST = _NPAD - 1
_BLK = 512
_GRID = _NPAD // _BLK

_sc_mesh = plsc.VectorSubcoreMesh(core_axis_name="c", subcore_axis_name="s")


def _deg_body(dst_hbm, ones_hbm, zeros_hbm, out_hbm, didx, buf, shared):
    c = lax.axis_index("c")
    s = lax.axis_index("s")
    pltpu.sync_copy(dst_hbm.at[c, s], didx)
    pltpu.sync_copy(zeros_hbm, buf)
    for k in range(_RPT // _C):
        pltpu.sync_copy(buf, shared.at[pl.ds(s * _RPT + k * _C, _C)])
    pltpu.sync_copy(ones_hbm, buf)
    plsc.subcore_barrier()

    def body(j, carry):
        pltpu.sync_copy(buf, shared.at[didx.at[j]], add=True)
        return carry

    lax.fori_loop(0, _K, body, 0)
    plsc.subcore_barrier()
    pltpu.sync_copy(shared.at[pl.ds(s * _RPT, _RPT)],
                    out_hbm.at[c, pl.ds(s * _RPT, _RPT)])


_sc_deg = pl.kernel(
    _deg_body,
    out_type=jax.ShapeDtypeStruct((_NC, _NPAD, _D), jnp.float32),
    mesh=_sc_mesh,
    scratch_types=[
        pltpu.VMEM((_K, _C), jnp.int32),
        pltpu.VMEM((_C, _D), jnp.float32),
        pltpu.VMEM_SHARED((_NPAD, _D), jnp.float32),
    ],
)


_NB = 2
_KH = _K // 2


def _agg_body(src_hbm, dst_hbm, y_hbm, out_hbm, sidx, didx,
              r0, r1, shared, g0, g1):
    c = lax.axis_index("c")
    s = lax.axis_index("s")
    rows = [r0, r1]
    gsem = [g0, g1]
    pltpu.sync_copy(src_hbm.at[c, s], sidx)
    pltpu.sync_copy(y_hbm.at[pl.ds(_NPAD - _C, _C)], r0)
    for k in range(_RPT // _C):
        pltpu.sync_copy(r0, shared.at[pl.ds(s * _RPT + k * _C, _C)])
    plsc.subcore_barrier()

    for b in range(_NB):
        pltpu.async_copy(y_hbm.at[sidx.at[b]], rows[b], gsem[b])

    for p in range(2):
        pltpu.sync_copy(dst_hbm.at[c, s, pl.ds(p * _KH, _KH)], didx)
        n_full = _KH // _NB - (1 if p == 1 else 0)

        def body(t, carry, p=p):
            for b in range(_NB):
                j = t * _NB + b
                g = p * _KH + j
                pltpu.make_async_copy(y_hbm.at[pl.ds(0, _C)], rows[b],
                                      gsem[b]).wait()
                pltpu.sync_copy(rows[b], shared.at[didx.at[j]], add=True)
                pltpu.async_copy(y_hbm.at[sidx.at[g + _NB]], rows[b], gsem[b])
            return carry

        lax.fori_loop(0, n_full, body, 0)

    for b in range(_NB):
        j = _KH - _NB + b
        pltpu.make_async_copy(y_hbm.at[pl.ds(0, _C)], rows[b], gsem[b]).wait()
        pltpu.sync_copy(rows[b], shared.at[didx.at[j]], add=True)
    plsc.subcore_barrier()
    pltpu.sync_copy(shared.at[pl.ds(s * _RPT, _RPT)],
                    out_hbm.at[c, pl.ds(s * _RPT, _RPT)])


_sc_agg = pl.kernel(
    _agg_body,
    out_type=jax.ShapeDtypeStruct((_NC, _NPAD, _D), jnp.float32),
    mesh=_sc_mesh,
    scratch_types=[
        pltpu.VMEM((_K, _C), jnp.int32),
        pltpu.VMEM((_KH, _C), jnp.int32),
        pltpu.VMEM((_C, _D), jnp.float32),
        pltpu.VMEM((_C, _D), jnp.float32),
        pltpu.VMEM_SHARED((_NPAD, _D), jnp.float32),
        pltpu.SemaphoreType.DMA,
        pltpu.SemaphoreType.DMA,
    ],
)


def _mm1_body(x_ref, w_ref, d0_ref, d1_ref, y_ref, dinv_ref):
    i = pl.program_id(0)
    deg = d0_ref[:, 0:1] + d1_ref[:, 0:1] + 1.0
    dinv = lax.rsqrt(deg)
    h = jnp.dot(x_ref[...], w_ref[...], preferred_element_type=jnp.float32)
    rows = i * _BLK + lax.broadcasted_iota(jnp.int32, (_BLK, _D), 0)
    y_ref[...] = jnp.where(rows < _N, h * dinv, 0.0)
    dinv_ref[...] = jnp.broadcast_to(dinv, (_BLK, 8))


def _mm2_body(p0_ref, p1_ref, y_ref, dinv_ref, b_ref, w_ref, y2_ref):
    i = pl.program_id(0)
    dinv = dinv_ref[:, 0:1]
    agg = p0_ref[0] + p1_ref[0] + y_ref[...]
    t = jnp.maximum(agg * dinv + b_ref[...], 0.0)
    h2 = jnp.dot(t, w_ref[...], preferred_element_type=jnp.float32)
    rows = i * _BLK + lax.broadcasted_iota(jnp.int32, (_BLK, _D), 0)
    y2_ref[...] = jnp.where(rows < _N, h2 * dinv, 0.0)


def _fin_body(q0_ref, q1_ref, y2_ref, dinv_ref, b_ref, out_ref):
    dinv = dinv_ref[:, 0:1]
    agg = q0_ref[0] + q1_ref[0] + y2_ref[...]
    out_ref[...] = agg * dinv + b_ref[...]


_row_spec = pl.BlockSpec((_BLK, _D), lambda i: (i, 0))
_deg_spec = pl.BlockSpec((1, _BLK, _D), lambda i: (0, i, 0))
_deg_spec1 = pl.BlockSpec((1, _BLK, _D), lambda i: (1, i, 0))
_w_spec = pl.BlockSpec((_D, _D), lambda i: (0, 0))
_b_spec = pl.BlockSpec((1, _D), lambda i: (0, 0))
_dinv_spec = pl.BlockSpec((_BLK, 8), lambda i: (i, 0))
_p_spec0 = pl.BlockSpec((1, _BLK, _D), lambda i: (0, i, 0))
_p_spec1 = pl.BlockSpec((1, _BLK, _D), lambda i: (1, i, 0))


def _mm1_deg_body(x_ref, w_ref, dp_ref0, dp_ref1, y_ref, dinv_ref):
    i = pl.program_id(0)
    deg = dp_ref0[0, :, 0:1] + dp_ref1[0, :, 0:1] + 1.0
    dinv = lax.rsqrt(deg)
    h = jnp.dot(x_ref[...], w_ref[...], preferred_element_type=jnp.float32)
    rows = i * _BLK + lax.broadcasted_iota(jnp.int32, (_BLK, _D), 0)
    y_ref[...] = jnp.where(rows < _N, h * dinv, 0.0)
    dinv_ref[...] = jnp.broadcast_to(dinv, (_BLK, 8))


_mm1 = pl.pallas_call(
    _mm1_deg_body,
    grid=(_GRID,),
    in_specs=[_row_spec, _w_spec, _deg_spec, _deg_spec1],
    out_specs=[_row_spec, _dinv_spec],
    out_shape=[
        jax.ShapeDtypeStruct((_NPAD, _D), jnp.float32),
        jax.ShapeDtypeStruct((_NPAD, 8), jnp.float32),
    ],
)

_mm2 = pl.pallas_call(
    _mm2_body,
    grid=(_GRID,),
    in_specs=[_p_spec0, _p_spec1, _row_spec, _dinv_spec, _b_spec, _w_spec],
    out_specs=_row_spec,
    out_shape=jax.ShapeDtypeStruct((_NPAD, _D), jnp.float32),
)

_fin = pl.pallas_call(
    _fin_body,
    grid=(_GRID,),
    in_specs=[_p_spec0, _p_spec1, _row_spec, _dinv_spec, _b_spec],
    out_specs=_row_spec,
    out_shape=jax.ShapeDtypeStruct((_NPAD, _D), jnp.float32),
)


@jax.jit
def _run(x, edge_index, W1, b1, W2, b2):
    src = edge_index[0].astype(jnp.int32)
    dst = edge_index[1].astype(jnp.int32)
    pads = jnp.full((_EPAD - _E,), _PAD_SRC, jnp.int32)
    padd = _NPAD - _C + (jnp.arange(_EPAD - _E, dtype=jnp.int32) % _C)
    src_p = (jnp.concatenate([src, pads]).reshape(_K, _NC, _NS, _C)
             .transpose(1, 2, 0, 3).reshape(_NC, _NS, _K, _C))
    dst_p = (jnp.concatenate([dst, padd]).reshape(_K, _NC, _NS, _C)
             .transpose(1, 2, 0, 3).reshape(_NC, _NS, _K, _C))
    x_p = jnp.zeros((_NPAD, _D), jnp.float32).at[:_N].set(x)

    ones_r = jnp.ones((_C, _D), jnp.float32)
    zeros_r = jnp.zeros((_C, _D), jnp.float32)
    degp = _sc_deg(dst_p, ones_r, zeros_r)
    y1, dinv = _mm1(x_p, W1, degp, degp)
    p = _sc_agg(src_p, dst_p, y1)
    y2 = _mm2(p, p, y1, dinv, b1.reshape(1, _D), W2)
    q = _sc_agg(src_p, dst_p, y2)
    out = _fin(q, q, y2, dinv, b2.reshape(1, _D))
    return out[:_N]


def kernel(x, edge_index, W1, b1, W2, b2):
    return _run(x, edge_index, W1, b1, W2, b2)


# speedup vs baseline: 1.0193x; 1.0193x over previous
import jax
import jax.numpy as jnp
from jax import lax
from jax.experimental import pallas as pl
from jax.experimental.pallas import tpu as pltpu
from jax.experimental.pallas import tpu_sc as plsc

_N = 10000
_E = 320000
_D = 128
_NC = 2
_NS = 16
_C = 128
_K = 80
_EPAD = _NC * _NS * _K * _C
_NPAD = 10240
_RPT = _NPAD // _NS
_PAD_SRC = _N
_PAD_DST = _NPAD - 1
_BLK = 512
_GRID = _NPAD // _BLK

_sc_mesh = plsc.VectorSubcoreMesh(core_axis_name="c", subcore_axis_name="s")


def _deg_body(dst_hbm, ones_hbm, zeros_hbm, out_hbm, didx, buf, shared):
    c = lax.axis_index("c")
    s = lax.axis_index("s")
    pltpu.sync_copy(dst_hbm.at[c, s], didx)
    pltpu.sync_copy(zeros_hbm, buf)
    for k in range(_RPT // _C):
        pltpu.sync_copy(buf, shared.at[pl.ds(s * _RPT + k * _C, _C)])
    pltpu.sync_copy(ones_hbm, buf)
    plsc.subcore_barrier()

    def body(j, carry):
        pltpu.sync_copy(buf, shared.at[didx.at[j]], add=True)
        return carry

    lax.fori_loop(0, _K, body, 0)
    plsc.subcore_barrier()
    pltpu.sync_copy(shared.at[pl.ds(s * _RPT, _RPT)],
                    out_hbm.at[c, pl.ds(s * _RPT, _RPT)])


_sc_deg = pl.kernel(
    _deg_body,
    out_type=jax.ShapeDtypeStruct((_NC, _NPAD, _D), jnp.float32),
    mesh=_sc_mesh,
    scratch_types=[
        pltpu.VMEM((_K, _C), jnp.int32),
        pltpu.VMEM((_C, _D), jnp.float32),
        pltpu.VMEM_SHARED((_NPAD, _D), jnp.float32),
    ],
)


_NB = 2
_KH = _K // 2


def _agg_body(src_hbm, dst_hbm, y_hbm, out_hbm, sidx, didx,
              r0, r1, shared, g0, g1):
    c = lax.axis_index("c")
    s = lax.axis_index("s")
    rows = [r0, r1]
    gsem = [g0, g1]
    pltpu.sync_copy(src_hbm.at[c, s], sidx)
    pltpu.sync_copy(y_hbm.at[pl.ds(_NPAD - _C, _C)], r0)
    for k in range(_RPT // _C):
        pltpu.sync_copy(r0, shared.at[pl.ds(s * _RPT + k * _C, _C)])
    plsc.subcore_barrier()

    for b in range(_NB):
        pltpu.async_copy(y_hbm.at[sidx.at[b]], rows[b], gsem[b])

    for p in range(2):
        pltpu.sync_copy(dst_hbm.at[c, s, pl.ds(p * _KH, _KH)], didx)
        n_full = _KH // _NB - (1 if p == 1 else 0)

        def body(t, carry, p=p):
            for b in range(_NB):
                j = t * _NB + b
                g = p * _KH + j
                pltpu.make_async_copy(y_hbm.at[pl.ds(0, _C)], rows[b],
                                      gsem[b]).wait()
                pltpu.sync_copy(rows[b], shared.at[didx.at[j]], add=True)
                pltpu.async_copy(y_hbm.at[sidx.at[g + _NB]], rows[b], gsem[b])
            return carry

        lax.fori_loop(0, n_full, body, 0)

    for b in range(_NB):
        j = _KH - _NB + b
        pltpu.make_async_copy(y_hbm.at[pl.ds(0, _C)], rows[b], gsem[b]).wait()
        pltpu.sync_copy(rows[b], shared.at[didx.at[j]], add=True)
    plsc.subcore_barrier()
    pltpu.sync_copy(shared.at[pl.ds(s * _RPT, _RPT)],
                    out_hbm.at[c, pl.ds(s * _RPT, _RPT)])


_sc_agg = pl.kernel(
    _agg_body,
    out_type=jax.ShapeDtypeStruct((_NC, _NPAD, _D), jnp.float32),
    mesh=_sc_mesh,
    scratch_types=[
        pltpu.VMEM((_K, _C), jnp.int32),
        pltpu.VMEM((_KH, _C), jnp.int32),
        pltpu.VMEM((_C, _D), jnp.float32),
        pltpu.VMEM((_C, _D), jnp.float32),
        pltpu.VMEM_SHARED((_NPAD, _D), jnp.float32),
        pltpu.SemaphoreType.DMA,
        pltpu.SemaphoreType.DMA,
    ],
)


def _mm1_body(x_ref, w_ref, d0_ref, d1_ref, y_ref, dinv_ref):
    i = pl.program_id(0)
    deg = d0_ref[:, 0:1] + d1_ref[:, 0:1] + 1.0
    dinv = lax.rsqrt(deg)
    h = jnp.dot(x_ref[...], w_ref[...], preferred_element_type=jnp.float32)
    rows = i * _BLK + lax.broadcasted_iota(jnp.int32, (_BLK, _D), 0)
    y_ref[...] = jnp.where(rows < _N, h * dinv, 0.0)
    dinv_ref[...] = jnp.broadcast_to(dinv, (_BLK, 8))


def _mm2_body(p0_ref, p1_ref, y_ref, dinv_ref, b_ref, w_ref, y2_ref):
    i = pl.program_id(0)
    dinv = dinv_ref[:, 0:1]
    agg = p0_ref[0] + p1_ref[0] + y_ref[...]
    t = jnp.maximum(agg * dinv + b_ref[...], 0.0)
    h2 = jnp.dot(t, w_ref[...], preferred_element_type=jnp.float32)
    rows = i * _BLK + lax.broadcasted_iota(jnp.int32, (_BLK, _D), 0)
    y2_ref[...] = jnp.where(rows < _N, h2 * dinv, 0.0)


def _fin_body(q0_ref, q1_ref, y2_ref, dinv_ref, b_ref, out_ref):
    dinv = dinv_ref[:, 0:1]
    agg = q0_ref[0] + q1_ref[0] + y2_ref[...]
    out_ref[...] = agg * dinv + b_ref[...]


_row_spec = pl.BlockSpec((_BLK, _D), lambda i: (i, 0))
_deg_spec = pl.BlockSpec((1, _BLK, _D), lambda i: (0, i, 0))
_deg_spec1 = pl.BlockSpec((1, _BLK, _D), lambda i: (1, i, 0))
_w_spec = pl.BlockSpec((_D, _D), lambda i: (0, 0))
_b_spec = pl.BlockSpec((1, _D), lambda i: (0, 0))
_dinv_spec = pl.BlockSpec((_BLK, 8), lambda i: (i, 0))
_p_spec0 = pl.BlockSpec((1, _BLK, _D), lambda i: (0, i, 0))
_p_spec1 = pl.BlockSpec((1, _BLK, _D), lambda i: (1, i, 0))


def _mm1_deg_body(x_ref, w_ref, dp_ref0, dp_ref1, y_ref, dinv_ref):
    i = pl.program_id(0)
    deg = dp_ref0[0, :, 0:1] + dp_ref1[0, :, 0:1] + 1.0
    dinv = lax.rsqrt(deg)
    h = jnp.dot(x_ref[...], w_ref[...], preferred_element_type=jnp.float32)
    rows = i * _BLK + lax.broadcasted_iota(jnp.int32, (_BLK, _D), 0)
    y_ref[...] = jnp.where(rows < _N, h * dinv, 0.0)
    dinv_ref[...] = jnp.broadcast_to(dinv, (_BLK, 8))


_mm1 = pl.pallas_call(
    _mm1_deg_body,
    grid=(_GRID,),
    in_specs=[_row_spec, _w_spec, _deg_spec, _deg_spec1],
    out_specs=[_row_spec, _dinv_spec],
    out_shape=[
        jax.ShapeDtypeStruct((_NPAD, _D), jnp.float32),
        jax.ShapeDtypeStruct((_NPAD, 8), jnp.float32),
    ],
)

_mm2 = pl.pallas_call(
    _mm2_body,
    grid=(_GRID,),
    in_specs=[_p_spec0, _p_spec1, _row_spec, _dinv_spec, _b_spec, _w_spec],
    out_specs=_row_spec,
    out_shape=jax.ShapeDtypeStruct((_NPAD, _D), jnp.float32),
)

_fin = pl.pallas_call(
    _fin_body,
    grid=(_GRID,),
    in_specs=[_p_spec0, _p_spec1, _row_spec, _dinv_spec, _b_spec],
    out_specs=_row_spec,
    out_shape=jax.ShapeDtypeStruct((_NPAD, _D), jnp.float32),
)


@jax.jit
def _run(x, edge_index, W1, b1, W2, b2):
    src = edge_index[0].astype(jnp.int32)
    dst = edge_index[1].astype(jnp.int32)
    pads = jnp.full((_EPAD - _E,), _PAD_SRC, jnp.int32)
    padd = _NPAD - _C + (jnp.arange(_EPAD - _E, dtype=jnp.int32) % _C)
    src_p = (jnp.concatenate([src, pads]).reshape(_NS * _K, _NC, _C)
             .transpose(1, 0, 2).reshape(_NC, _NS, _K, _C))
    dst_p = (jnp.concatenate([dst, padd]).reshape(_NS * _K, _NC, _C)
             .transpose(1, 0, 2).reshape(_NC, _NS, _K, _C))
    x_p = jnp.zeros((_NPAD, _D), jnp.float32).at[:_N].set(x)

    ones_r = jnp.ones((_C, _D), jnp.float32)
    zeros_r = jnp.zeros((_C, _D), jnp.float32)
    degp = _sc_deg(dst_p, ones_r, zeros_r)
    y1, dinv = _mm1(x_p, W1, degp, degp)
    p = _sc_agg(src_p, dst_p, y1)
    y2 = _mm2(p, p, y1, dinv, b1.reshape(1, _D), W2)
    q = _sc_agg(src_p, dst_p, y2)
    out = _fin(q, q, y2, dinv, b2.reshape(1, _D))
    return out[:_N]


def kernel(x, edge_index, W1, b1, W2, b2):
    return _run(x, edge_index, W1, b1, W2, b2)
